# Initial kernel scaffold; baseline (speedup 1.0000x reference)
import jax
import jax.numpy as jnp
from jax.experimental import pallas as pl


def kernel(x, batch, condition, sW1, sb1, sW2, sb2, g1eps, g1W1, g1b1, g1W2, g1b2, g2eps, g2W1, g2b1, g2W2, g2b2, oW1, ob1, oW2, ob2):
    raise NotImplementedError("write your pallas kernel here")


# speedup vs baseline: 11.5139x; 11.5139x over previous
import functools

import jax
import jax.numpy as jnp
from jax import lax
from jax.experimental import pallas as pl
from jax.experimental.pallas import tpu as pltpu

K = 15


def _bmm(a, b):
    return jnp.dot(a.astype(jnp.bfloat16), b.astype(jnp.bfloat16),
                   preferred_element_type=jnp.float32)


def _split3(x):
    f32, bf16 = jnp.float32, jnp.bfloat16
    hi = x.astype(bf16).astype(f32)
    r1 = x - hi
    mid = r1.astype(bf16).astype(f32)
    lo = r1 - mid
    return hi, mid, lo


def _graph_body(x_ref, sW1, sb1, sW2, sb2, g1W1, g1b1, g1W2, g1b2,
                g2W1, g2b1, g2W2, g2b2, oW1, ob1, oW2, ob2, eps_ref,
                xout_ref, xemb_ref, idx_ref, *, nodes):
    f32 = jnp.float32
    bf16 = jnp.bfloat16

    def ffn(v, W1, b1, W2, b2):
        h = jnp.maximum(_bmm(v, W1[...]) + b1[...], 0.0)
        return _bmm(h, W2[...]) + b2[...]

    x = x_ref[...]
    y = ffn(x, sW1, sb1, sW2, sb2)

    p = y[:, :3]
    q0 = p[:, 0:1] * p[:, 0:1]
    q1 = p[:, 1:2] * p[:, 1:2]
    q2 = p[:, 2:3] * p[:, 2:3]
    sn = (q0 + q2) + q1

    row = lax.broadcasted_iota(jnp.int32, (nodes, nodes), 0)
    col = lax.broadcasted_iota(jnp.int32, (nodes, nodes), 1)
    diag = row == col
    eye_b = diag.astype(bf16)

    snT = None
    for piece in _split3(sn):
        t = lax.dot_general(piece.astype(bf16), eye_b,
                            (((0,), (0,)), ((), ())),
                            preferred_element_type=f32)
        snT = t if snT is None else snT + t

    pb = p.astype(bf16)
    G = lax.dot_general(pb, pb, (((1,), (1,)), ((), ())),
                        preferred_element_type=f32)
    d2 = (sn + snT) - 2.0 * G
    d2 = jnp.where(diag, d2 + 1e9, d2)

    col16 = lax.broadcasted_iota(jnp.int32, (nodes, 16), 1)
    adj = jnp.zeros((nodes, nodes), f32)
    idxm = jnp.zeros((nodes, 16), jnp.int32)
    d = d2
    for k in range(K):
        m = jnp.min(d, axis=1, keepdims=True)
        cand = jnp.where(d == m, col, jnp.int32(2**30))
        a = jnp.min(cand, axis=1, keepdims=True)
        sel = col == a
        adj = adj + sel.astype(f32)
        d = jnp.where(sel, jnp.inf, d)
        idxm = jnp.where(col16 == k, a, idxm)
    idx_ref[...] = idxm

    adj_b = adj.astype(bf16)

    def agg(v):
        acc = None
        for piece in _split3(v):
            t = lax.dot_general(adj_b, piece.astype(bf16),
                                (((1,), (0,)), ((), ())),
                                preferred_element_type=f32)
            acc = t if acc is None else acc + t
        return acc

    eps1 = eps_ref[0, 0]
    eps2 = eps_ref[0, 1]

    z1 = y + ffn((1.0 + eps1) * y + agg(y), g1W1, g1b1, g1W2, g1b2)
    z2 = z1 + ffn((1.0 + eps2) * z1 + agg(z1), g2W1, g2b1, g2W2, g2b2)

    xout_ref[...] = z2
    xemb_ref[...] = ffn(z2, oW1, ob1, oW2, ob2)


def kernel(x, batch, condition, sW1, sb1, sW2, sb2,
           g1eps, g1W1, g1b1, g1W2, g1b2,
           g2eps, g2W1, g2b1, g2W2, g2b2,
           oW1, ob1, oW2, ob2):
    n = x.shape[0]
    n_graphs = condition.shape[0]
    nodes = n // n_graphs
    f32 = jnp.float32

    eps = jnp.stack([g1eps, g2eps]).reshape(1, 2).astype(f32)

    def row2(a):
        return a.reshape(1, -1).astype(f32)

    full = lambda a: pl.BlockSpec(a.shape, lambda g: (0, 0))
    grid_spec = pl.GridSpec(
        grid=(n_graphs,),
        in_specs=[
            pl.BlockSpec((nodes, x.shape[1]), lambda g: (g, 0)),
            full(sW1), full(row2(sb1)), full(sW2), full(row2(sb2)),
            full(g1W1), full(row2(g1b1)), full(g1W2), full(row2(g1b2)),
            full(g2W1), full(row2(g2b1)), full(g2W2), full(row2(g2b2)),
            full(oW1), full(row2(ob1)), full(oW2), full(row2(ob2)),
            pl.BlockSpec((1, 2), lambda g: (0, 0), memory_space=pltpu.SMEM),
        ],
        out_specs=[
            pl.BlockSpec((nodes, 64), lambda g: (g, 0)),
            pl.BlockSpec((nodes, 128), lambda g: (g, 0)),
            pl.BlockSpec((nodes, 16), lambda g: (g, 0)),
        ],
    )
    xout, xemb, idx = pl.pallas_call(
        functools.partial(_graph_body, nodes=nodes),
        grid_spec=grid_spec,
        out_shape=[
            jax.ShapeDtypeStruct((n, 64), f32),
            jax.ShapeDtypeStruct((n, 128), f32),
            jax.ShapeDtypeStruct((n, 16), jnp.int32),
        ],
        compiler_params=pltpu.CompilerParams(
            dimension_semantics=("arbitrary",),
        ),
    )(x, sW1, row2(sb1), sW2, row2(sb2),
      g1W1, row2(g1b1), g1W2, row2(g1b2),
      g2W1, row2(g2b1), g2W2, row2(g2b2),
      oW1, row2(ob1), oW2, row2(ob2), eps)

    offs = (jnp.arange(n_graphs, dtype=jnp.int32) * nodes)[:, None, None]
    src = (idx[:, :K].reshape(n_graphs, nodes, K) + offs).reshape(-1)
    dst = jnp.repeat(jnp.arange(n, dtype=jnp.int32), K)
    ei = jnp.stack([src, dst]).astype(jnp.int64)
    return (xout, xemb, ei)


# speedup vs baseline: 16.0780x; 1.3964x over previous
import functools

import jax
import jax.numpy as jnp
from jax import lax
from jax.experimental import pallas as pl
from jax.experimental.pallas import tpu as pltpu

K = 15


def _bmm(a, b):
    return jnp.dot(a.astype(jnp.bfloat16), b.astype(jnp.bfloat16),
                   preferred_element_type=jnp.float32)


def _split3(x):
    f32, bf16 = jnp.float32, jnp.bfloat16
    hi = x.astype(bf16).astype(f32)
    r1 = x - hi
    mid = r1.astype(bf16).astype(f32)
    lo = r1 - mid
    return hi, mid, lo


def _graph_body(x_ref, sW1, sb1, sW2, sb2, g1W1, g1b1, g1W2, g1b2,
                g2W1, g2b1, g2W2, g2b2, oW1, ob1, oW2, ob2,
                eye_b_ref, E_ref, colf_ref, eps_ref,
                xout_ref, xemb_ref, idx_ref, *, nodes):
    f32 = jnp.float32
    bf16 = jnp.bfloat16

    def ffn(v, W1, b1, W2, b2):
        h = jnp.maximum(_bmm(v, W1[...]) + b1[...], 0.0)
        return _bmm(h, W2[...]) + b2[...]

    x = x_ref[...]
    y = ffn(x, sW1, sb1, sW2, sb2)

    p = y[:, :3]
    q0 = p[:, 0:1] * p[:, 0:1]
    q1 = p[:, 1:2] * p[:, 1:2]
    q2 = p[:, 2:3] * p[:, 2:3]
    sn = (q0 + q2) + q1

    eye_b = eye_b_ref[...]

    snT = None
    for piece in _split3(sn):
        t = lax.dot_general(piece.astype(bf16), eye_b,
                            (((0,), (0,)), ((), ())),
                            preferred_element_type=f32)
        snT = t if snT is None else snT + t

    pb = p.astype(bf16)
    G = lax.dot_general(pb, pb, (((1,), (1,)), ((), ())),
                        preferred_element_type=f32)
    d2 = (sn + snT) - 2.0 * G
    d2 = d2 + E_ref[...]

    colf = colf_ref[...]
    col16 = lax.broadcasted_iota(jnp.int32, (nodes, 16), 1)
    idxm = jnp.zeros((nodes, 16), jnp.int32)
    d = d2
    for k in range(K):
        m = jnp.min(d, axis=1, keepdims=True)
        cand = jnp.where(d == m, colf, jnp.inf)
        a = jnp.min(cand, axis=1, keepdims=True)
        d = jnp.where(colf == a, jnp.inf, d)
        idxm = jnp.where(col16 == k, a.astype(jnp.int32), idxm)
    idx_ref[...] = idxm

    adj_b = jnp.where(jnp.isinf(d), f32(1), f32(0)).astype(bf16)

    def agg(v):
        hi, mid, _ = _split3(v)
        t1 = lax.dot_general(adj_b, hi.astype(bf16), (((1,), (0,)), ((), ())),
                             preferred_element_type=f32)
        t2 = lax.dot_general(adj_b, mid.astype(bf16), (((1,), (0,)), ((), ())),
                             preferred_element_type=f32)
        return t1 + t2

    eps1 = eps_ref[0, 0]
    eps2 = eps_ref[0, 1]

    z1 = y + ffn((1.0 + eps1) * y + agg(y), g1W1, g1b1, g1W2, g1b2)
    z2 = z1 + ffn((1.0 + eps2) * z1 + agg(z1), g2W1, g2b1, g2W2, g2b2)

    xout_ref[...] = z2
    xemb_ref[...] = ffn(z2, oW1, ob1, oW2, ob2)


def kernel(x, batch, condition, sW1, sb1, sW2, sb2,
           g1eps, g1W1, g1b1, g1W2, g1b2,
           g2eps, g2W1, g2b1, g2W2, g2b2,
           oW1, ob1, oW2, ob2):
    n = x.shape[0]
    n_graphs = condition.shape[0]
    nodes = n // n_graphs
    f32 = jnp.float32

    eps = jnp.stack([g1eps, g2eps]).reshape(1, 2).astype(f32)

    def row2(a):
        return a.reshape(1, -1).astype(f32)

    full = lambda a: pl.BlockSpec(a.shape, lambda g: (0, 0))
    grid_spec = pl.GridSpec(
        grid=(n_graphs,),
        in_specs=[
            pl.BlockSpec((nodes, x.shape[1]), lambda g: (g, 0)),
            full(sW1), full(row2(sb1)), full(sW2), full(row2(sb2)),
            full(g1W1), full(row2(g1b1)), full(g1W2), full(row2(g1b2)),
            full(g2W1), full(row2(g2b1)), full(g2W2), full(row2(g2b2)),
            full(oW1), full(row2(ob1)), full(oW2), full(row2(ob2)),
            pl.BlockSpec((nodes, nodes), lambda g: (0, 0)),
            pl.BlockSpec((nodes, nodes), lambda g: (0, 0)),
            pl.BlockSpec((1, nodes), lambda g: (0, 0)),
            pl.BlockSpec((1, 2), lambda g: (0, 0), memory_space=pltpu.SMEM),
        ],
        out_specs=[
            pl.BlockSpec((nodes, 64), lambda g: (g, 0)),
            pl.BlockSpec((nodes, 128), lambda g: (g, 0)),
            pl.BlockSpec((nodes, 16), lambda g: (g, 0)),
        ],
    )
    xout, xemb, idx = pl.pallas_call(
        functools.partial(_graph_body, nodes=nodes),
        grid_spec=grid_spec,
        out_shape=[
            jax.ShapeDtypeStruct((n, 64), f32),
            jax.ShapeDtypeStruct((n, 128), f32),
            jax.ShapeDtypeStruct((n, 16), jnp.int32),
        ],
        compiler_params=pltpu.CompilerParams(
            dimension_semantics=("arbitrary",),
        ),
    )(x, sW1, row2(sb1), sW2, row2(sb2),
      g1W1, row2(g1b1), g1W2, row2(g1b2),
      g2W1, row2(g2b1), g2W2, row2(g2b2),
      oW1, row2(ob1), oW2, row2(ob2),
      jnp.eye(nodes, dtype=jnp.bfloat16),
      jnp.eye(nodes, dtype=f32) * 1e9,
      jnp.arange(nodes, dtype=f32).reshape(1, nodes), eps)

    offs = (jnp.arange(n_graphs, dtype=jnp.int32) * nodes)[:, None, None]
    src = (idx[:, :K].reshape(n_graphs, nodes, K) + offs).reshape(-1)
    dst = jnp.repeat(jnp.arange(n, dtype=jnp.int32), K)
    ei = jnp.stack([src, dst]).astype(jnp.int64)
    return (xout, xemb, ei)


# speedup vs baseline: 16.1346x; 1.0035x over previous
import functools

import jax
import jax.numpy as jnp
from jax import lax
from jax.experimental import pallas as pl
from jax.experimental.pallas import tpu as pltpu

K = 15


def _bmm(a, b):
    return jnp.dot(a.astype(jnp.bfloat16), b.astype(jnp.bfloat16),
                   preferred_element_type=jnp.float32)


def _split3(x):
    f32, bf16 = jnp.float32, jnp.bfloat16
    hi = x.astype(bf16).astype(f32)
    r1 = x - hi
    mid = r1.astype(bf16).astype(f32)
    lo = r1 - mid
    return hi, mid, lo


def _graph_body(x_ref, sW1, sb1, sW2, sb2, g1W1, g1b1, g1W2, g1b2,
                g2W1, g2b1, g2W2, g2b2, oW1, ob1, oW2, ob2,
                eye_b_ref, colf_ref, eps_ref,
                xout_ref, xemb_ref, idx_ref, *, nodes):
    f32 = jnp.float32
    bf16 = jnp.bfloat16

    def ffn(v, W1, b1, W2, b2):
        h = jnp.maximum(_bmm(v, W1[...]) + b1[...], 0.0)
        return _bmm(h, W2[...]) + b2[...]

    x = x_ref[...]
    y = ffn(x, sW1, sb1, sW2, sb2)

    p = y[:, :3]
    q0 = p[:, 0:1] * p[:, 0:1]
    q1 = p[:, 1:2] * p[:, 1:2]
    q2 = p[:, 2:3] * p[:, 2:3]
    sn = (q0 + q2) + q1

    eye_b = eye_b_ref[...]

    snT = None
    for piece in _split3(sn):
        t = lax.dot_general(piece.astype(bf16), eye_b,
                            (((0,), (0,)), ((), ())),
                            preferred_element_type=f32)
        snT = t if snT is None else snT + t

    pb = p.astype(bf16)
    G = lax.dot_general(pb, pb, (((1,), (1,)), ((), ())),
                        preferred_element_type=f32)
    d2 = (sn + snT) - 2.0 * G
    row = lax.broadcasted_iota(jnp.int32, (nodes, nodes), 0)
    coli = lax.broadcasted_iota(jnp.int32, (nodes, nodes), 1)
    d2 = jnp.where(row == coli, d2 + 1e9, d2)

    colf = colf_ref[...]
    col16 = lax.broadcasted_iota(jnp.int32, (nodes, 16), 1)
    idxm = jnp.zeros((nodes, 16), jnp.int32)
    d = d2
    for k in range(K):
        m = jnp.min(d, axis=1, keepdims=True)
        cand = jnp.where(d == m, colf, jnp.inf)
        a = jnp.min(cand, axis=1, keepdims=True)
        d = jnp.where(colf == a, jnp.inf, d)
        idxm = jnp.where(col16 == k, a.astype(jnp.int32), idxm)
    idx_ref[...] = idxm

    adj_b = jnp.where(jnp.isinf(d), f32(1), f32(0)).astype(bf16)

    def agg(v):
        hi, mid, _ = _split3(v)
        t1 = lax.dot_general(adj_b, hi.astype(bf16), (((1,), (0,)), ((), ())),
                             preferred_element_type=f32)
        t2 = lax.dot_general(adj_b, mid.astype(bf16), (((1,), (0,)), ((), ())),
                             preferred_element_type=f32)
        return t1 + t2

    eps1 = eps_ref[0, 0]
    eps2 = eps_ref[0, 1]

    z1 = y + ffn((1.0 + eps1) * y + agg(y), g1W1, g1b1, g1W2, g1b2)
    z2 = z1 + ffn((1.0 + eps2) * z1 + agg(z1), g2W1, g2b1, g2W2, g2b2)

    xout_ref[...] = z2
    xemb_ref[...] = ffn(z2, oW1, ob1, oW2, ob2)


def kernel(x, batch, condition, sW1, sb1, sW2, sb2,
           g1eps, g1W1, g1b1, g1W2, g1b2,
           g2eps, g2W1, g2b1, g2W2, g2b2,
           oW1, ob1, oW2, ob2):
    n = x.shape[0]
    n_graphs = condition.shape[0]
    nodes = n // n_graphs
    f32 = jnp.float32

    eps = jnp.stack([g1eps, g2eps]).reshape(1, 2).astype(f32)

    def row2(a):
        return a.reshape(1, -1).astype(f32)

    full = lambda a: pl.BlockSpec(a.shape, lambda g: (0, 0))
    grid_spec = pl.GridSpec(
        grid=(n_graphs,),
        in_specs=[
            pl.BlockSpec((nodes, x.shape[1]), lambda g: (g, 0)),
            full(sW1), full(row2(sb1)), full(sW2), full(row2(sb2)),
            full(g1W1), full(row2(g1b1)), full(g1W2), full(row2(g1b2)),
            full(g2W1), full(row2(g2b1)), full(g2W2), full(row2(g2b2)),
            full(oW1), full(row2(ob1)), full(oW2), full(row2(ob2)),
            pl.BlockSpec((nodes, nodes), lambda g: (0, 0)),
            pl.BlockSpec((1, nodes), lambda g: (0, 0)),
            pl.BlockSpec((1, 2), lambda g: (0, 0), memory_space=pltpu.SMEM),
        ],
        out_specs=[
            pl.BlockSpec((nodes, 64), lambda g: (g, 0)),
            pl.BlockSpec((nodes, 128), lambda g: (g, 0)),
            pl.BlockSpec((nodes, 16), lambda g: (g, 0)),
        ],
    )
    xout, xemb, idx = pl.pallas_call(
        functools.partial(_graph_body, nodes=nodes),
        grid_spec=grid_spec,
        out_shape=[
            jax.ShapeDtypeStruct((n, 64), f32),
            jax.ShapeDtypeStruct((n, 128), f32),
            jax.ShapeDtypeStruct((n, 16), jnp.int32),
        ],
        compiler_params=pltpu.CompilerParams(
            dimension_semantics=("arbitrary",),
        ),
    )(x, sW1, row2(sb1), sW2, row2(sb2),
      g1W1, row2(g1b1), g1W2, row2(g1b2),
      g2W1, row2(g2b1), g2W2, row2(g2b2),
      oW1, row2(ob1), oW2, row2(ob2),
      jnp.eye(nodes, dtype=jnp.bfloat16),
      jnp.arange(nodes, dtype=f32).reshape(1, nodes), eps)

    offs = (jnp.arange(n_graphs, dtype=jnp.int32) * nodes)[:, None, None]
    src = (idx[:, :K].reshape(n_graphs, nodes, K) + offs).reshape(-1)
    dst = jnp.repeat(jnp.arange(n, dtype=jnp.int32), K)
    ei = jnp.stack([src, dst]).astype(jnp.int64)
    return (xout, xemb, ei)
